# baseline (speedup 1.0000x reference)
import numpy as np
import jax
import jax.numpy as jnp
from jax.experimental import pallas as pl
from jax.experimental.pallas import tpu as pltpu

_K = 8192
_D = 64
_BLK = 1024
_NSTEP = _K // _BLK


def _uniform_transposed() -> np.ndarray:
    n = _K * _D
    with np.errstate(over="ignore"):
        x1 = np.arange(n, dtype=np.uint32)
        x0 = np.zeros(n, dtype=np.uint32)
        k0, k1 = np.uint32(0), np.uint32(42)
        ks = [k0, k1, np.uint32(int(k0) ^ int(k1) ^ 0x1BD11BDA)]
        x0 = x0 + ks[0]
        x1 = x1 + ks[1]
        rot = ((13, 15, 26, 6), (17, 29, 16, 24))
        for g in range(5):
            for r in rot[g % 2]:
                x0 = (x0 + x1).astype(np.uint32)
                x1 = ((x1 << np.uint32(r)) | (x1 >> np.uint32(32 - r))).astype(np.uint32)
                x1 = x1 ^ x0
            x0 = (x0 + ks[(g + 1) % 3]).astype(np.uint32)
            x1 = (x1 + ks[(g + 2) % 3] + np.uint32(g + 1)).astype(np.uint32)
        bits = x0 ^ x1
        fbits = (bits >> np.uint32(9)) | np.uint32(0x3F800000)
        u = (fbits.view(np.float32) - np.float32(1.0)) + np.float32(1e-20)
    ut = u.reshape(_K, _D).T
    return np.ascontiguousarray(
        ut.reshape(_D, _NSTEP, _BLK).transpose(1, 0, 2))


_U_T = _uniform_transposed()


def _pure_index_body(w_hbm, u_hbm, og_ref, idx_ref,
                     wbuf, ubuf, m_acc, i_acc, wsem, usem):
    def wcopy(j, slot):
        return pltpu.make_async_copy(
            w_hbm.at[pl.ds(j * _BLK, _BLK), :], wbuf.at[slot], wsem.at[slot])

    def ucopy(j, slot):
        return pltpu.make_async_copy(u_hbm.at[j], ubuf.at[slot], usem.at[slot])

    wcopy(0, 0).start()
    ucopy(0, 0).start()
    for j in range(_NSTEP):
        slot = j & 1
        if j + 1 < _NSTEP:
            wcopy(j + 1, 1 - slot).start()
            ucopy(j + 1, 1 - slot).start()
        wcopy(j, slot).wait()
        ucopy(j, slot).wait()
        wt = wbuf[slot].T
        u = ubuf[slot]
        z = wt - jnp.log(-jnp.log(u))
        zmax = jnp.max(z, axis=0, keepdims=True)
        e = jnp.exp(z - zmax)
        s = jnp.sum(e, axis=0, keepdims=True)
        y = e / s
        m = jnp.float32(1.0) / s
        c = jax.lax.broadcasted_iota(jnp.int32, (_D, _BLK), 0)
        idx = jnp.min(jnp.where(y == m, c, jnp.int32(_D)), axis=0, keepdims=True)
        lo = j * _BLK
        m_acc[:, pl.ds(lo, _BLK)] = (jnp.float32(1.0) - m) + m
        i_acc[:, pl.ds(lo, _BLK)] = idx

    og_ref[pl.ds(0, 1)] = jnp.full((1,), 1.0, jnp.float32)
    og_ref[pl.ds(1, _K)] = m_acc[0, :]
    og_ref[pl.ds(_K + 1, 1)] = jnp.full((1,), 1.0, jnp.float32)
    idx_ref[pl.ds(0, 1)] = jnp.full((1,), _D, jnp.int32)
    idx_ref[pl.ds(1, _K)] = i_acc[0, :]
    idx_ref[pl.ds(_K + 1, 1)] = jnp.full((1,), _D + 1, jnp.int32)


def kernel(image, W, step):
    del image, step
    u = jnp.asarray(_U_T)
    _any = pl.BlockSpec(memory_space=pltpu.MemorySpace.HBM)
    _vmem = pl.BlockSpec(memory_space=pltpu.MemorySpace.VMEM)
    og, idx = pl.pallas_call(
        _pure_index_body,
        in_specs=[_any, _any],
        out_specs=[_vmem, _vmem],
        out_shape=[
            jax.ShapeDtypeStruct((_K + 2,), jnp.float32),
            jax.ShapeDtypeStruct((_K + 2,), jnp.int32),
        ],
        scratch_shapes=[
            pltpu.VMEM((2, _BLK, _D), jnp.float32),
            pltpu.VMEM((2, _D, _BLK), jnp.float32),
            pltpu.VMEM((1, _K), jnp.float32),
            pltpu.VMEM((1, _K), jnp.int32),
            pltpu.SemaphoreType.DMA((2,)),
            pltpu.SemaphoreType.DMA((2,)),
        ],
    )(W, u)
    return (og, idx)


# speedup vs baseline: 2.1983x; 2.1983x over previous
import numpy as np
import jax
import jax.numpy as jnp
from jax.experimental import pallas as pl
from jax.experimental.pallas import tpu as pltpu

_K = 8192
_D = 64
_BLK = 1024
_NSTEP = _K // _BLK


def _uniform_transposed() -> np.ndarray:
    n = _K * _D
    with np.errstate(over="ignore"):
        x1 = np.arange(n, dtype=np.uint32)
        x0 = np.zeros(n, dtype=np.uint32)
        k0, k1 = np.uint32(0), np.uint32(42)
        ks = [k0, k1, np.uint32(int(k0) ^ int(k1) ^ 0x1BD11BDA)]
        x0 = x0 + ks[0]
        x1 = x1 + ks[1]
        rot = ((13, 15, 26, 6), (17, 29, 16, 24))
        for g in range(5):
            for r in rot[g % 2]:
                x0 = (x0 + x1).astype(np.uint32)
                x1 = ((x1 << np.uint32(r)) | (x1 >> np.uint32(32 - r))).astype(np.uint32)
                x1 = x1 ^ x0
            x0 = (x0 + ks[(g + 1) % 3]).astype(np.uint32)
            x1 = (x1 + ks[(g + 2) % 3] + np.uint32(g + 1)).astype(np.uint32)
        bits = x0 ^ x1
        fbits = (bits >> np.uint32(9)) | np.uint32(0x3F800000)
        u = (fbits.view(np.float32) - np.float32(1.0)) + np.float32(1e-20)
    return np.ascontiguousarray(u.reshape(_K, _D).T)


_U_T = _uniform_transposed()


def _pure_index_body(wt_ref, u_ref, og_ref, idx_ref):
    for j in range(_NSTEP):
        lo = j * _BLK
        wt = wt_ref[:, pl.ds(lo, _BLK)]
        u = u_ref[:, pl.ds(lo, _BLK)]
        z = wt - jnp.log(-jnp.log(u))
        zmax = jnp.max(z, axis=0, keepdims=True)
        e = jnp.exp(z - zmax)
        s = jnp.sum(e, axis=0, keepdims=True)
        y = e / s
        m = jnp.float32(1.0) / s
        c = jax.lax.broadcasted_iota(jnp.int32, (_D, _BLK), 0)
        idx = jnp.min(jnp.where(y == m, c, jnp.int32(_D)), axis=0, keepdims=True)
        og_ref[pl.ds(1 + lo, _BLK)] = ((jnp.float32(1.0) - m) + m)[0]
        idx_ref[pl.ds(1 + lo, _BLK)] = idx[0]

    og_ref[pl.ds(0, 1)] = jnp.full((1,), 1.0, jnp.float32)
    og_ref[pl.ds(_K + 1, 1)] = jnp.full((1,), 1.0, jnp.float32)
    idx_ref[pl.ds(0, 1)] = jnp.full((1,), _D, jnp.int32)
    idx_ref[pl.ds(_K + 1, 1)] = jnp.full((1,), _D + 1, jnp.int32)


def kernel(image, W, step):
    del image, step
    u = jnp.asarray(_U_T)
    _vmem = pl.BlockSpec(memory_space=pltpu.MemorySpace.VMEM)
    og, idx = pl.pallas_call(
        _pure_index_body,
        in_specs=[_vmem, _vmem],
        out_specs=[_vmem, _vmem],
        out_shape=[
            jax.ShapeDtypeStruct((_K + 2,), jnp.float32),
            jax.ShapeDtypeStruct((_K + 2,), jnp.int32),
        ],
    )(W.T, u)
    return (og, idx)


# speedup vs baseline: 2.2292x; 1.0141x over previous
import numpy as np
import jax
import jax.numpy as jnp
from jax.experimental import pallas as pl
from jax.experimental.pallas import tpu as pltpu

_K = 8192
_D = 64
_BLK = 1024
_NSTEP = _K // _BLK


def _uniform_transposed() -> np.ndarray:
    n = _K * _D
    with np.errstate(over="ignore"):
        x1 = np.arange(n, dtype=np.uint32)
        x0 = np.zeros(n, dtype=np.uint32)
        k0, k1 = np.uint32(0), np.uint32(42)
        ks = [k0, k1, np.uint32(int(k0) ^ int(k1) ^ 0x1BD11BDA)]
        x0 = x0 + ks[0]
        x1 = x1 + ks[1]
        rot = ((13, 15, 26, 6), (17, 29, 16, 24))
        for g in range(5):
            for r in rot[g % 2]:
                x0 = (x0 + x1).astype(np.uint32)
                x1 = ((x1 << np.uint32(r)) | (x1 >> np.uint32(32 - r))).astype(np.uint32)
                x1 = x1 ^ x0
            x0 = (x0 + ks[(g + 1) % 3]).astype(np.uint32)
            x1 = (x1 + ks[(g + 2) % 3] + np.uint32(g + 1)).astype(np.uint32)
        bits = x0 ^ x1
        fbits = (bits >> np.uint32(9)) | np.uint32(0x3F800000)
        u = (fbits.view(np.float32) - np.float32(1.0)) + np.float32(1e-20)
    return np.ascontiguousarray(u.reshape(_K, _D).T)


_U_T = _uniform_transposed()


def _pure_index_body(wt_ref, u_ref, og_ref, idx_ref):
    c = jax.lax.broadcasted_iota(
        jnp.int32, (_D, _BLK), 0).astype(jnp.float32)
    for j in range(_NSTEP):
        lo = j * _BLK
        wt = wt_ref[:, pl.ds(lo, _BLK)]
        u = u_ref[:, pl.ds(lo, _BLK)]
        z = wt - jnp.log(-jnp.log(u))
        zmax = jnp.max(z, axis=0, keepdims=True)
        e = jnp.exp(z - zmax)
        s = jnp.sum(e, axis=0, keepdims=True)
        y = e / s
        m = jnp.float32(1.0) / s
        idxf = jnp.min(jnp.where(y == m, c, jnp.float32(_D)), axis=0,
                       keepdims=True)
        og_ref[pl.ds(1 + lo, _BLK)] = ((jnp.float32(1.0) - m) + m)[0]
        idx_ref[pl.ds(1 + lo, _BLK)] = idxf[0].astype(jnp.int32)

    og_ref[pl.ds(0, 1)] = jnp.full((1,), 1.0, jnp.float32)
    og_ref[pl.ds(_K + 1, 1)] = jnp.full((1,), 1.0, jnp.float32)
    idx_ref[pl.ds(0, 1)] = jnp.full((1,), _D, jnp.int32)
    idx_ref[pl.ds(_K + 1, 1)] = jnp.full((1,), _D + 1, jnp.int32)


def kernel(image, W, step):
    del image, step
    u = jnp.asarray(_U_T)
    _vmem = pl.BlockSpec(memory_space=pltpu.MemorySpace.VMEM)
    og, idx = pl.pallas_call(
        _pure_index_body,
        in_specs=[_vmem, _vmem],
        out_specs=[_vmem, _vmem],
        out_shape=[
            jax.ShapeDtypeStruct((_K + 2,), jnp.float32),
            jax.ShapeDtypeStruct((_K + 2,), jnp.int32),
        ],
    )(W.T, u)
    return (og, idx)
